# baseline (speedup 1.0000x reference)
import functools
import jax
import jax.numpy as jnp
from jax import lax
from jax.experimental import pallas as pl
from jax.experimental.pallas import tpu as pltpu
from jax.experimental.pallas import tpu_sc as plsc

N = 10000
E = 320000
NC = 2
NS = 16
CH = 80
EW = E // (NC * NS)
NCHUNK = EW // CH
RPT = 624
ZB = 104


def _ln(y, g, b, eps=1e-5):
    mu = jnp.mean(y, axis=-1, keepdims=True)
    var = jnp.mean((y - mu) ** 2, axis=-1, keepdims=True)
    return (y - mu) * lax.rsqrt(var + eps) * g + b


CPB = 25
NBLK = NCHUNK // CPB


def _make_sc_mp(F, FV=None, with_deg=False):
    if FV is None:
        FV = F
    mesh = plsc.VectorSubcoreMesh(core_axis_name="c", subcore_axis_name="s")
    out_type = [jax.ShapeDtypeStruct((NC, N, F), jnp.float32)]
    scratch = [
        pltpu.VMEM_SHARED((N, F), jnp.float32),
        pltpu.VMEM((2, CPB, CH), jnp.int32),
        pltpu.VMEM((2, CPB, CH), jnp.int32),
        pltpu.VMEM((2, CPB, CH), jnp.float32),
        pltpu.VMEM((2, CH, F), jnp.float32),
        pltpu.VMEM((16, F), jnp.float32),
        pltpu.SemaphoreType.DMA,
        pltpu.SemaphoreType.DMA,
        pltpu.SemaphoreType.DMA,
        pltpu.SemaphoreType.DMA,
        pltpu.SemaphoreType.DMA,
    ]
    if with_deg:
        out_type.append(jax.ShapeDtypeStruct((NC, N, 16), jnp.float32))
        scratch += [
            pltpu.VMEM_SHARED((N, 16), jnp.float32),
            pltpu.VMEM((CH, 16), jnp.float32),
            pltpu.VMEM((16, 16), jnp.float32),
            pltpu.SemaphoreType.DMA,
        ]

    def body(x_hbm, src_hbm, dst_hbm, dist_hbm, part_hbm, *rest):
        if with_deg:
            (dpart_hbm, acc, srcb, dstb, distb, rows, zbuf,
             sg0, sg1, ss0, ss1, si, dacc, ones_v, zbufd, sd) = rest
        else:
            (acc, srcb, dstb, distb, rows, zbuf,
             sg0, sg1, ss0, ss1, si) = rest
        cid = lax.axis_index("c")
        sid = lax.axis_index("s")
        zeros16 = jnp.zeros((16,), jnp.float32)
        sg = (sg0, sg1)
        ss = (ss0, ss1)

        def zrow(r, _):
            for k in range(F // 16):
                zbuf[r, pl.ds(k * 16, 16)] = zeros16
            if with_deg:
                zbufd[r, pl.ds(0, 16)] = zeros16
                ones_v[r, pl.ds(0, 16)] = jnp.ones((16,), jnp.float32)
                ones_v[r + 16, pl.ds(0, 16)] = jnp.ones((16,), jnp.float32)
                ones_v[r + 32, pl.ds(0, 16)] = jnp.ones((16,), jnp.float32)
                ones_v[r + 48, pl.ds(0, 16)] = jnp.ones((16,), jnp.float32)
                ones_v[r + 64, pl.ds(0, 16)] = jnp.ones((16,), jnp.float32)
            return 0
        lax.fori_loop(0, 16, zrow, 0)

        row0 = sid * RPT
        for i in range(RPT // 16):
            pltpu.sync_copy(zbuf, acc.at[pl.ds(row0 + i * 16, 16)])
            if with_deg:
                pltpu.sync_copy(zbufd, dacc.at[pl.ds(row0 + i * 16, 16)])

        @pl.when(sid == NS - 1)
        def _():
            pltpu.sync_copy(zbuf, acc.at[pl.ds(NS * RPT, 16)])
            if with_deg:
                pltpu.sync_copy(zbufd, dacc.at[pl.ds(NS * RPT, 16)])

        blk0 = (cid * NS + sid) * NBLK

        def issue_idx(b, slot):
            d0 = pltpu.async_copy(src_hbm.at[blk0 + b], srcb.at[slot], si)
            d1 = pltpu.async_copy(dst_hbm.at[blk0 + b], dstb.at[slot], si)
            d2 = pltpu.async_copy(dist_hbm.at[blk0 + b], distb.at[slot], si)
            return (d0, d1, d2)

        def gather(slot, c, buf):
            pltpu.async_copy(x_hbm.at[srcb.at[slot, c]], rows.at[buf], sg[buf])

        def wait_gather(buf):
            pltpu.make_async_copy(x_hbm.at[pl.ds(0, CH)], rows.at[buf],
                                  sg[buf]).wait()

        def scatter(slot, c, buf):
            pltpu.async_copy(rows.at[buf], acc.at[dstb.at[slot, c]],
                             ss[buf], add=True)

        def wait_scatter(buf):
            pltpu.make_async_copy(x_hbm.at[pl.ds(0, CH)], rows.at[buf],
                                  ss[buf]).wait()

        if with_deg:
            def deg_scatter(slot, c):
                pltpu.async_copy(ones_v, dacc.at[dstb.at[slot, c]],
                                 sd, add=True)

            def wait_deg():
                pltpu.make_async_copy(dpart_hbm.at[0, pl.ds(0, CH)],
                                      ones_v, sd).wait()

        def mul(slot, c, buf):
            def group(g, _):
                d16 = distb[slot, c, pl.ds(g * 16, 16)]
                for j in range(16):
                    dj = d16[j]
                    r = g * 16 + j
                    for k in range(FV // 16):
                        rows[buf, r, pl.ds(k * 16, 16)] = (
                            rows[buf, r, pl.ds(k * 16, 16)] * dj)
                return 0
            lax.fori_loop(0, CH // 16, group, 0)

        descs = issue_idx(0, 0)
        plsc.subcore_barrier()

        for b in range(NBLK):
            s = b % 2
            for d in descs:
                d.wait()
            if b + 1 < NBLK:
                descs = issue_idx(b + 1, (b + 1) % 2)

            gather(s, 0, 0)

            def pair(p, _):
                c0 = 2 * p
                gather(s, c0 + 1, 1)
                wait_gather(0)
                mul(s, c0, 0)
                scatter(s, c0, 0)
                if with_deg:
                    deg_scatter(s, c0)
                wait_gather(1)
                mul(s, c0 + 1, 1)
                scatter(s, c0 + 1, 1)
                if with_deg:
                    deg_scatter(s, c0 + 1)
                wait_scatter(0)
                gather(s, c0 + 2, 0)
                if with_deg:
                    wait_deg()
                    wait_deg()
                wait_scatter(1)
                return 0
            lax.fori_loop(0, CPB // 2, pair, 0)

            wait_gather(0)
            mul(s, CPB - 1, 0)
            scatter(s, CPB - 1, 0)
            if with_deg:
                deg_scatter(s, CPB - 1)
                wait_deg()
            wait_scatter(0)

        plsc.subcore_barrier()

        pltpu.sync_copy(acc.at[pl.ds(row0, RPT)],
                        part_hbm.at[cid, pl.ds(row0, RPT)])
        if with_deg:
            pltpu.sync_copy(dacc.at[pl.ds(row0, RPT)],
                            dpart_hbm.at[cid, pl.ds(row0, RPT)])

        @pl.when(sid == NS - 1)
        def _():
            pltpu.sync_copy(acc.at[pl.ds(NS * RPT, 16)],
                            part_hbm.at[cid, pl.ds(NS * RPT, 16)])
            if with_deg:
                pltpu.sync_copy(dacc.at[pl.ds(NS * RPT, 16)],
                                dpart_hbm.at[cid, pl.ds(NS * RPT, 16)])

    return pl.kernel(body, out_type=tuple(out_type), mesh=mesh,
                     scratch_types=scratch,
                     compiler_params=pltpu.CompilerParams(
                         use_tc_tiling_on_sc=False))


def _make_sc_mp_sp(F):
    mesh = plsc.VectorSubcoreMesh(core_axis_name="c", subcore_axis_name="s")
    out_type = jax.ShapeDtypeStruct((NC, N, F), jnp.float32)
    scratch = [
        pltpu.VMEM_SHARED((N, F), jnp.float32),
        pltpu.VMEM_SHARED((N, F), jnp.float32),
        pltpu.VMEM((2, CPB, CH), jnp.int32),
        pltpu.VMEM((2, CPB, CH), jnp.int32),
        pltpu.VMEM((2, CPB, CH), jnp.float32),
        pltpu.VMEM((2, CH, F), jnp.float32),
        pltpu.VMEM((16, F), jnp.float32),
        pltpu.SemaphoreType.DMA,
        pltpu.SemaphoreType.DMA,
        pltpu.SemaphoreType.DMA,
        pltpu.SemaphoreType.DMA,
        pltpu.SemaphoreType.DMA,
    ]

    def body(x_hbm, src_hbm, dst_hbm, dist_hbm, part_hbm,
             xsp, acc, srcb, dstb, distb, rows, zbuf,
             sg0, sg1, ss0, ss1, si):
        cid = lax.axis_index("c")
        sid = lax.axis_index("s")
        zeros16 = jnp.zeros((16,), jnp.float32)
        sg = (sg0, sg1)
        ss = (ss0, ss1)

        def zrow(r, _):
            for k in range(F // 16):
                zbuf[r, pl.ds(k * 16, 16)] = zeros16
            return 0
        lax.fori_loop(0, 16, zrow, 0)

        row0 = sid * RPT
        pltpu.sync_copy(x_hbm.at[pl.ds(row0, RPT)], xsp.at[pl.ds(row0, RPT)])
        for i in range(RPT // 16):
            pltpu.sync_copy(zbuf, acc.at[pl.ds(row0 + i * 16, 16)])

        @pl.when(sid == NS - 1)
        def _():
            pltpu.sync_copy(x_hbm.at[pl.ds(NS * RPT, 16)],
                            xsp.at[pl.ds(NS * RPT, 16)])
            pltpu.sync_copy(zbuf, acc.at[pl.ds(NS * RPT, 16)])

        blk0 = (cid * NS + sid) * NBLK

        def issue_idx(b, slot):
            d0 = pltpu.async_copy(src_hbm.at[blk0 + b], srcb.at[slot], si)
            d1 = pltpu.async_copy(dst_hbm.at[blk0 + b], dstb.at[slot], si)
            d2 = pltpu.async_copy(dist_hbm.at[blk0 + b], distb.at[slot], si)
            return (d0, d1, d2)

        def gather(slot, c, buf):
            pltpu.async_copy(xsp.at[srcb.at[slot, c]], rows.at[buf], sg[buf])

        def wait_gather(buf):
            pltpu.make_async_copy(x_hbm.at[pl.ds(0, CH)], rows.at[buf],
                                  sg[buf]).wait()

        def scatter(slot, c, buf):
            pltpu.async_copy(rows.at[buf], acc.at[dstb.at[slot, c]],
                             ss[buf], add=True)

        def wait_scatter(buf):
            pltpu.make_async_copy(x_hbm.at[pl.ds(0, CH)], rows.at[buf],
                                  ss[buf]).wait()

        def mul(slot, c, buf):
            def group(g, _):
                d16 = distb[slot, c, pl.ds(g * 16, 16)]
                for j in range(16):
                    dj = d16[j]
                    r = g * 16 + j
                    for k in range(F // 16):
                        rows[buf, r, pl.ds(k * 16, 16)] = (
                            rows[buf, r, pl.ds(k * 16, 16)] * dj)
                return 0
            lax.fori_loop(0, CH // 16, group, 0)

        descs = issue_idx(0, 0)
        plsc.subcore_barrier()

        for b in range(NBLK):
            s = b % 2
            for d in descs:
                d.wait()
            if b + 1 < NBLK:
                descs = issue_idx(b + 1, (b + 1) % 2)

            gather(s, 0, 0)

            def pair(p, _):
                c0 = 2 * p
                gather(s, c0 + 1, 1)
                wait_gather(0)
                mul(s, c0, 0)
                scatter(s, c0, 0)
                wait_gather(1)
                mul(s, c0 + 1, 1)
                scatter(s, c0 + 1, 1)
                wait_scatter(0)
                gather(s, c0 + 2, 0)
                wait_scatter(1)
                return 0
            lax.fori_loop(0, CPB // 2, pair, 0)

            wait_gather(0)
            mul(s, CPB - 1, 0)
            scatter(s, CPB - 1, 0)
            wait_scatter(0)

        plsc.subcore_barrier()

        pltpu.sync_copy(acc.at[pl.ds(row0, RPT)],
                        part_hbm.at[cid, pl.ds(row0, RPT)])

        @pl.when(sid == NS - 1)
        def _():
            pltpu.sync_copy(acc.at[pl.ds(NS * RPT, 16)],
                            part_hbm.at[cid, pl.ds(NS * RPT, 16)])

    return pl.kernel(body, out_type=out_type, mesh=mesh, scratch_types=scratch,
                     compiler_params=pltpu.CompilerParams(
                         use_tc_tiling_on_sc=False))


_sc_mp128 = _make_sc_mp(128, with_deg=True)
_sc_mp64 = _make_sc_mp(128, 64)
_sc_mp64_sp = _make_sc_mp_sp(64)


def _proj_body(h_ref, wp0_ref, bp0_ref, gp0_ref, bep0_ref,
               wp1_ref, bp1_ref, gp1_ref, bep1_ref, out_ref):
    h = h_ref[...]
    dn = (((1,), (1,)), ((), ()))
    y0 = lax.dot_general(h[:, :64], wp0_ref[...], dn,
                         preferred_element_type=jnp.float32) + bp0_ref[...]
    y1 = lax.dot_general(h[:, 64:], wp1_ref[...], dn,
                         preferred_element_type=jnp.float32) + bp1_ref[...]
    p0 = _ln(y0, gp0_ref[...], bep0_ref[...])
    p1 = _ln(y1, gp1_ref[...], bep1_ref[...])
    out_ref[:, :64] = jnp.where(p0 > 0, p0, 0.01 * p0)
    out_ref[:, 64:] = jnp.where(p1 > 0, p1, 0.01 * p1)


def _combine_body(x_ref, part_ref, dpart_ref, w_ref, b_ref, g_ref, be_ref,
                  out_ref, *, F, O, ln_relu, pad_out):
    deg = dpart_ref[0, :, 0:1] + dpart_ref[1, :, 0:1]
    norm = jnp.where(deg > 0, 1.0 / deg, 0.0)
    ah = (part_ref[0][:, :F] + part_ref[1][:, :F]) * norm
    w = w_ref[...]
    dn = (((1,), (1,)), ((), ()))
    y = (lax.dot_general(x_ref[:, :F], w[:, :F], dn,
                         preferred_element_type=jnp.float32)
         + lax.dot_general(ah, w[:, F:], dn,
                           preferred_element_type=jnp.float32)
         + b_ref[...])
    if ln_relu:
        y = _ln(y, g_ref[...], be_ref[...])
        y = jnp.maximum(y, 0.0)
    if pad_out:
        out_ref[:, :O] = y
        out_ref[:, O:] = jnp.zeros_like(out_ref[:, O:])
    else:
        out_ref[...] = y


def _tc_combine(x, parts, dparts, W, b, g, be, F, O, ln_relu, pad_out):
    body = functools.partial(_combine_body, F=F, O=O, ln_relu=ln_relu,
                             pad_out=pad_out)
    return pl.pallas_call(
        body,
        out_shape=jax.ShapeDtypeStruct((N, 128 if pad_out else O),
                                       jnp.float32),
    )(x, parts, dparts, W, b, g, be)


@jax.jit
def kernel(h, edge_index, edge_dist, Wp0, bp0, gp0, bep0, Wp1, bp1, gp1, bep1,
           W0, b0, g0, be0, W1, b1, g1, be1, W2, b2):
    src = edge_index[0]
    dst = edge_index[1]
    dist = edge_dist.reshape(E)
    blks = (NC * NS * NBLK, CPB, CH)
    src3 = src.reshape(blks)
    dst3 = dst.reshape(blks)
    dist3 = dist.reshape(blks)

    x = pl.pallas_call(
        _proj_body,
        out_shape=jax.ShapeDtypeStruct((N, 128), jnp.float32),
    )(h, Wp0, bp0, gp0, bep0, Wp1, bp1, gp1, bep1)

    part0, dpart = _sc_mp128(x, src3, dst3, dist3)
    x = _tc_combine(x, part0, dpart, W0, b0, g0, be0, 128, 64, True, False)

    part1 = _sc_mp64_sp(x, src3, dst3, dist3)
    x = _tc_combine(x, part1, dpart, W1, b1, g1, be1, 64, 64, True, False)

    part2 = _sc_mp64_sp(x, src3, dst3, dist3)
    out = _tc_combine(x, part2, dpart, W2, b2, b2, b2, 64, 40, False, False)
    return out


# speedup vs baseline: 1.1774x; 1.1774x over previous
import functools
import jax
import jax.numpy as jnp
from jax import lax
from jax.experimental import pallas as pl
from jax.experimental.pallas import tpu as pltpu
from jax.experimental.pallas import tpu_sc as plsc

N = 10000
E = 320000
NC = 2
NS = 16
CH = 80
EW = E // (NC * NS)
NCHUNK = EW // CH
RPT = 624
ZB = 104


def _ln(y, g, b, eps=1e-5):
    mu = jnp.mean(y, axis=-1, keepdims=True)
    var = jnp.mean((y - mu) ** 2, axis=-1, keepdims=True)
    return (y - mu) * lax.rsqrt(var + eps) * g + b


CPB = 25
NBLK = NCHUNK // CPB


def _make_sc_mp(F, FV=None, with_deg=False):
    if FV is None:
        FV = F
    mesh = plsc.VectorSubcoreMesh(core_axis_name="c", subcore_axis_name="s")
    out_type = [jax.ShapeDtypeStruct((NC, N, F), jnp.float32)]
    scratch = [
        pltpu.VMEM_SHARED((N, F), jnp.float32),
        pltpu.VMEM((2, CPB, CH), jnp.int32),
        pltpu.VMEM((2, CPB, CH), jnp.int32),
        pltpu.VMEM((2, CPB, CH), jnp.float32),
        pltpu.VMEM((2, CH, F), jnp.float32),
        pltpu.VMEM((16, F), jnp.float32),
        pltpu.SemaphoreType.DMA,
        pltpu.SemaphoreType.DMA,
        pltpu.SemaphoreType.DMA,
        pltpu.SemaphoreType.DMA,
        pltpu.SemaphoreType.DMA,
    ]
    if with_deg:
        out_type.append(jax.ShapeDtypeStruct((NC, N, 16), jnp.float32))
        scratch += [
            pltpu.VMEM_SHARED((N, 16), jnp.float32),
            pltpu.VMEM((CH, 16), jnp.float32),
            pltpu.VMEM((16, 16), jnp.float32),
            pltpu.SemaphoreType.DMA,
        ]

    def body(x_hbm, src_hbm, dst_hbm, dist_hbm, part_hbm, *rest):
        if with_deg:
            (dpart_hbm, acc, srcb, dstb, distb, rows, zbuf,
             sg0, sg1, ss0, ss1, si, dacc, ones_v, zbufd, sd) = rest
        else:
            (acc, srcb, dstb, distb, rows, zbuf,
             sg0, sg1, ss0, ss1, si) = rest
        cid = lax.axis_index("c")
        sid = lax.axis_index("s")
        zeros16 = jnp.zeros((16,), jnp.float32)
        sg = (sg0, sg1)
        ss = (ss0, ss1)

        def zrow(r, _):
            for k in range(F // 16):
                zbuf[r, pl.ds(k * 16, 16)] = zeros16
            if with_deg:
                zbufd[r, pl.ds(0, 16)] = zeros16
                ones_v[r, pl.ds(0, 16)] = jnp.ones((16,), jnp.float32)
                ones_v[r + 16, pl.ds(0, 16)] = jnp.ones((16,), jnp.float32)
                ones_v[r + 32, pl.ds(0, 16)] = jnp.ones((16,), jnp.float32)
                ones_v[r + 48, pl.ds(0, 16)] = jnp.ones((16,), jnp.float32)
                ones_v[r + 64, pl.ds(0, 16)] = jnp.ones((16,), jnp.float32)
            return 0
        lax.fori_loop(0, 16, zrow, 0)

        row0 = sid * RPT
        for i in range(RPT // 16):
            pltpu.sync_copy(zbuf, acc.at[pl.ds(row0 + i * 16, 16)])
            if with_deg:
                pltpu.sync_copy(zbufd, dacc.at[pl.ds(row0 + i * 16, 16)])

        @pl.when(sid == NS - 1)
        def _():
            pltpu.sync_copy(zbuf, acc.at[pl.ds(NS * RPT, 16)])
            if with_deg:
                pltpu.sync_copy(zbufd, dacc.at[pl.ds(NS * RPT, 16)])

        blk0 = (cid * NS + sid) * NBLK

        def issue_idx(b, slot):
            d0 = pltpu.async_copy(src_hbm.at[blk0 + b], srcb.at[slot], si)
            d1 = pltpu.async_copy(dst_hbm.at[blk0 + b], dstb.at[slot], si)
            d2 = pltpu.async_copy(dist_hbm.at[blk0 + b], distb.at[slot], si)
            return (d0, d1, d2)

        def gather(slot, c, buf):
            pltpu.async_copy(x_hbm.at[srcb.at[slot, c]], rows.at[buf], sg[buf])

        def wait_gather(buf):
            pltpu.make_async_copy(x_hbm.at[pl.ds(0, CH)], rows.at[buf],
                                  sg[buf]).wait()

        def scatter(slot, c, buf):
            pltpu.async_copy(rows.at[buf], acc.at[dstb.at[slot, c]],
                             ss[buf], add=True)

        def wait_scatter(buf):
            pltpu.make_async_copy(x_hbm.at[pl.ds(0, CH)], rows.at[buf],
                                  ss[buf]).wait()

        if with_deg:
            def deg_scatter(slot, c):
                pltpu.async_copy(ones_v, dacc.at[dstb.at[slot, c]],
                                 sd, add=True)

            def wait_deg():
                pltpu.make_async_copy(dpart_hbm.at[0, pl.ds(0, CH)],
                                      ones_v, sd).wait()

        def mul(slot, c, buf):
            def group(g, _):
                d16 = distb[slot, c, pl.ds(g * 16, 16)]
                for j in range(16):
                    dj = d16[j]
                    r = g * 16 + j
                    for k in range(FV // 16):
                        rows[buf, r, pl.ds(k * 16, 16)] = (
                            rows[buf, r, pl.ds(k * 16, 16)] * dj)
                return 0
            lax.fori_loop(0, CH // 16, group, 0)

        descs = issue_idx(0, 0)
        plsc.subcore_barrier()

        for b in range(NBLK):
            s = b % 2
            for d in descs:
                d.wait()
            if b + 1 < NBLK:
                descs = issue_idx(b + 1, (b + 1) % 2)

            gather(s, 0, 0)

            def pair(p, _):
                c0 = 2 * p
                gather(s, c0 + 1, 1)
                wait_gather(0)
                mul(s, c0, 0)
                scatter(s, c0, 0)
                if with_deg:
                    deg_scatter(s, c0)
                wait_gather(1)
                mul(s, c0 + 1, 1)
                scatter(s, c0 + 1, 1)
                if with_deg:
                    deg_scatter(s, c0 + 1)
                wait_scatter(0)
                gather(s, c0 + 2, 0)
                if with_deg:
                    wait_deg()
                    wait_deg()
                wait_scatter(1)
                return 0
            lax.fori_loop(0, CPB // 2, pair, 0)

            wait_gather(0)
            mul(s, CPB - 1, 0)
            scatter(s, CPB - 1, 0)
            if with_deg:
                deg_scatter(s, CPB - 1)
                wait_deg()
            wait_scatter(0)

        plsc.subcore_barrier()

        pltpu.sync_copy(acc.at[pl.ds(row0, RPT)],
                        part_hbm.at[cid, pl.ds(row0, RPT)])
        if with_deg:
            pltpu.sync_copy(dacc.at[pl.ds(row0, RPT)],
                            dpart_hbm.at[cid, pl.ds(row0, RPT)])

        @pl.when(sid == NS - 1)
        def _():
            pltpu.sync_copy(acc.at[pl.ds(NS * RPT, 16)],
                            part_hbm.at[cid, pl.ds(NS * RPT, 16)])
            if with_deg:
                pltpu.sync_copy(dacc.at[pl.ds(NS * RPT, 16)],
                                dpart_hbm.at[cid, pl.ds(NS * RPT, 16)])

    return pl.kernel(body, out_type=tuple(out_type), mesh=mesh,
                     scratch_types=scratch,
                     compiler_params=pltpu.CompilerParams(
                         use_tc_tiling_on_sc=False))


_sc_mp128 = _make_sc_mp(128, with_deg=True)
_sc_mp64 = _make_sc_mp(128, 64)


def _proj_body(h_ref, wp0_ref, bp0_ref, gp0_ref, bep0_ref,
               wp1_ref, bp1_ref, gp1_ref, bep1_ref, out_ref):
    h = h_ref[...]
    dn = (((1,), (1,)), ((), ()))
    y0 = lax.dot_general(h[:, :64], wp0_ref[...], dn,
                         preferred_element_type=jnp.float32) + bp0_ref[...]
    y1 = lax.dot_general(h[:, 64:], wp1_ref[...], dn,
                         preferred_element_type=jnp.float32) + bp1_ref[...]
    p0 = _ln(y0, gp0_ref[...], bep0_ref[...])
    p1 = _ln(y1, gp1_ref[...], bep1_ref[...])
    out_ref[:, :64] = jnp.where(p0 > 0, p0, 0.01 * p0)
    out_ref[:, 64:] = jnp.where(p1 > 0, p1, 0.01 * p1)


def _combine_body(x_ref, part_ref, dpart_ref, w_ref, b_ref, g_ref, be_ref,
                  out_ref, *, F, O, ln_relu, pad_out):
    deg = dpart_ref[0, :, 0:1] + dpart_ref[1, :, 0:1]
    norm = jnp.where(deg > 0, 1.0 / deg, 0.0)
    ah = (part_ref[0][:, :F] + part_ref[1][:, :F]) * norm
    w = w_ref[...]
    dn = (((1,), (1,)), ((), ()))
    y = (lax.dot_general(x_ref[:, :F], w[:, :F], dn,
                         preferred_element_type=jnp.float32)
         + lax.dot_general(ah, w[:, F:], dn,
                           preferred_element_type=jnp.float32)
         + b_ref[...])
    if ln_relu:
        y = _ln(y, g_ref[...], be_ref[...])
        y = jnp.maximum(y, 0.0)
    if pad_out:
        out_ref[:, :O] = y
        out_ref[:, O:] = jnp.zeros_like(out_ref[:, O:])
    else:
        out_ref[...] = y


def _tc_combine(x, parts, dparts, W, b, g, be, F, O, ln_relu, pad_out):
    body = functools.partial(_combine_body, F=F, O=O, ln_relu=ln_relu,
                             pad_out=pad_out)
    return pl.pallas_call(
        body,
        out_shape=jax.ShapeDtypeStruct((N, 128 if pad_out else O),
                                       jnp.float32),
    )(x, parts, dparts, W, b, g, be)


@jax.jit
def kernel(h, edge_index, edge_dist, Wp0, bp0, gp0, bep0, Wp1, bp1, gp1, bep1,
           W0, b0, g0, be0, W1, b1, g1, be1, W2, b2):
    src = edge_index[0]
    dst = edge_index[1]
    dist = edge_dist.reshape(E)
    blks = (NC * NS * NBLK, CPB, CH)
    src3 = src.reshape(blks)
    dst3 = dst.reshape(blks)
    dist3 = dist.reshape(blks)

    x = pl.pallas_call(
        _proj_body,
        out_shape=jax.ShapeDtypeStruct((N, 128), jnp.float32),
    )(h, Wp0, bp0, gp0, bep0, Wp1, bp1, gp1, bep1)

    part0, dpart = _sc_mp128(x, src3, dst3, dist3)
    x = _tc_combine(x, part0, dpart, W0, b0, g0, be0, 128, 64, True, True)

    (part1,) = _sc_mp64(x, src3, dst3, dist3)
    x = _tc_combine(x, part1, dpart, W1, b1, g1, be1, 64, 64, True, True)

    (part2,) = _sc_mp64(x, src3, dst3, dist3)
    out = _tc_combine(x, part2, dpart, W2, b2, b2, b2, 64, 40, False, False)
    return out
